# baseline (speedup 1.0000x reference)
import functools

import jax
import jax.numpy as jnp
from jax import lax
from jax.experimental import pallas as pl
from jax.experimental.pallas import tpu as pltpu
from jax.experimental.pallas import tpu_sc as plsc

B = 4096
L = 200
DIM = 64
V = 1000000

NC = 2
NS = 16
NW = NC * NS

V_FULL = 999936
CA = 256
N_CHUNKS = V_FULL // CA
TAIL = V - V_FULL


def _tr_body(tT_hbm, tail_hbm, out_hbm, in_b, ob, tail_b, sem):
    wid = lax.axis_index("s") * NC + lax.axis_index("c")
    n_chunks = jnp.where(wid < N_CHUNKS - (N_CHUNKS // NW) * NW,
                         N_CHUNKS // NW + 1, N_CHUNKS // NW)
    lane = lax.iota(jnp.int32, 16)

    rowpat = tuple(lax.rem(cb * 16 + lane, 64) for cb in range(8))
    colpat = tuple(lax.div(cb * 16 + lane, 64) for cb in range(8))

    def chunk(i, _):
        c0 = pl.multiple_of((wid + i * NW) * CA, 128)
        pltpu.sync_copy(tT_hbm.at[:, pl.ds(c0, CA)], in_b)

        def p_body(p, _):
            twop = p * 2
            vals = [plsc.load_gather(in_b, [rowpat[cb], colpat[cb] + twop])
                    for cb in range(8)]
            for cb in range(8):
                ob[p, pl.ds(cb * 16, 16)] = vals[cb]
            return 0

        lax.fori_loop(0, CA // 2, p_body, 0, unroll=4)

        pltpu.sync_copy(ob, out_hbm.at[pl.ds(lax.div(c0, 2), CA // 2)])
        return 0

    lax.fori_loop(0, n_chunks, chunk, 0)

    @pl.when(wid == 0)
    def _():
        pltpu.sync_copy(tail_hbm, tail_b)
        pltpu.sync_copy(tail_b, out_hbm.at[pl.ds(V_FULL // 2, TAIL // 2)])


N = B * L
N_W = N // NW
UNITS_W = N_W // 128


def _ga_body(idx_hbm, table_hbm, out_hbm, idx_v, idx2_v, base_v,
             pairs_b, out_b, sem):
    wid = lax.axis_index("s") * NC + lax.axis_index("c")
    base = wid * N_W
    pltpu.sync_copy(idx_hbm.at[pl.ds(base, N_W)], idx_v)
    lane = lax.iota(jnp.int32, 16)
    jrow = tuple(k * 16 + lane for k in range(8))

    def unit(u_local, _):
        u = wid * UNITS_W + u_local
        l = lax.div(u, 32)
        bt = lax.rem(u, 32)
        o = pl.multiple_of(u_local * 128, 128)
        for k in range(8):
            v = idx_v[pl.ds(o + k * 16, 16)]
            idx2_v[pl.ds(k * 16, 16)] = lax.shift_right_logical(v, 1)
            base_v[pl.ds(k * 16, 16)] = lax.shift_left(
                lax.bitwise_and(v, 1), 6)
        pltpu.async_copy(table_hbm.at[idx2_v], pairs_b, sem).wait()
        hb = tuple(base_v[pl.ds(k * 16, 16)] for k in range(8))

        def d_body(d, _):
            vals = [plsc.load_gather(pairs_b, [jrow[jb], hb[jb] + d])
                    for jb in range(8)]
            for jb in range(8):
                out_b[d, pl.ds(jb * 16, 16)] = vals[jb]
            return 0

        lax.fori_loop(0, DIM, d_body, 0, unroll=4)

        pltpu.sync_copy(
            out_b, out_hbm.at[l, :, pl.ds(pl.multiple_of(bt * 128, 128), 128)])
        return 0

    lax.fori_loop(0, UNITS_W, unit, 0)


@jax.jit
def _emb2(xt_flat, table_t, tail_p):
    mesh = plsc.VectorSubcoreMesh(core_axis_name="c", subcore_axis_name="s")
    tr = functools.partial(
        pl.kernel,
        mesh=mesh,
        out_type=jax.ShapeDtypeStruct((V // 2, 2 * DIM), jnp.float32),
        scratch_types=[
            pltpu.VMEM((DIM, CA), jnp.float32),
            pltpu.VMEM((CA // 2, 2 * DIM), jnp.float32),
            pltpu.VMEM((TAIL // 2, 2 * DIM), jnp.float32),
            pltpu.SemaphoreType.DMA,
        ],
        compiler_params=pltpu.CompilerParams(needs_layout_passes=False),
    )(_tr_body)
    table_rm = tr(table_t, tail_p)

    ga = functools.partial(
        pl.kernel,
        mesh=mesh,
        out_type=jax.ShapeDtypeStruct((L, DIM, B), jnp.float32),
        scratch_types=[
            pltpu.VMEM((N_W,), jnp.int32),
            pltpu.VMEM((128,), jnp.int32),
            pltpu.VMEM((128,), jnp.int32),
            pltpu.VMEM((128, 2 * DIM), jnp.float32),
            pltpu.VMEM((DIM, 128), jnp.float32),
            pltpu.SemaphoreType.DMA,
        ],
        compiler_params=pltpu.CompilerParams(needs_layout_passes=False),
    )(_ga_body)
    return ga(xt_flat, table_rm)


def kernel(x, table):
    xt_flat = x.T.reshape(N).astype(jnp.int32)
    tail_p = table[V_FULL:].reshape(TAIL // 2, 2 * DIM)
    out_p = _emb2(xt_flat, table.T, tail_p)
    return jnp.transpose(out_p, (2, 0, 1))


# speedup vs baseline: 4.0779x; 4.0779x over previous
import functools

import jax
import jax.numpy as jnp
from jax import lax
from jax.experimental import pallas as pl
from jax.experimental.pallas import tpu as pltpu
from jax.experimental.pallas import tpu_sc as plsc

B = 4096
L = 200
DIM = 64
V = 1000000

NC = 2
NS = 16
NW = NC * NS

V_FULL = 999936
CA = 256
N_CHUNKS = V_FULL // CA
TAIL = V - V_FULL


def _tr_body(tT_hbm, tail_hbm, out_hbm, in_b, ob, tail_b, sem):
    wid = lax.axis_index("s") * NC + lax.axis_index("c")
    n_chunks = jnp.where(wid < N_CHUNKS - (N_CHUNKS // NW) * NW,
                         N_CHUNKS // NW + 1, N_CHUNKS // NW)
    lane = lax.iota(jnp.int32, 16)

    rowpat = tuple(lax.rem(cb * 16 + lane, 64) for cb in range(8))
    colpat = tuple(lax.div(cb * 16 + lane, 64) for cb in range(8))

    def chunk(i, _):
        c0 = pl.multiple_of((wid + i * NW) * CA, 128)
        pltpu.sync_copy(tT_hbm.at[:, pl.ds(c0, CA)], in_b)

        @functools.partial(plsc.parallel_loop, 0, CA // 2, unroll=1)
        def p_body(p):
            twop = p * 2
            vals = [plsc.load_gather(in_b, [rowpat[cb], colpat[cb] + twop])
                    for cb in range(8)]
            for cb in range(8):
                ob[p, pl.ds(cb * 16, 16)] = vals[cb]

        pltpu.sync_copy(ob, out_hbm.at[pl.ds(lax.div(c0, 2), CA // 2)])
        return 0

    lax.fori_loop(0, n_chunks, chunk, 0)

    @pl.when(wid == 0)
    def _():
        pltpu.sync_copy(tail_hbm, tail_b)
        pltpu.sync_copy(tail_b, out_hbm.at[pl.ds(V_FULL // 2, TAIL // 2)])


N = B * L
N_W = N // NW
UNITS_W = N_W // 128


def _ga_body(idx_hbm, table_hbm, out_hbm, idx_v, idx2_v, base_v,
             pairs_b, out_b, sem):
    wid = lax.axis_index("s") * NC + lax.axis_index("c")
    base = wid * N_W
    pltpu.sync_copy(idx_hbm.at[pl.ds(base, N_W)], idx_v)
    lane = lax.iota(jnp.int32, 16)
    jrow = tuple(k * 16 + lane for k in range(8))

    def unit(u_local, _):
        u = wid * UNITS_W + u_local
        l = lax.div(u, 32)
        bt = lax.rem(u, 32)
        o = pl.multiple_of(u_local * 128, 128)
        for k in range(8):
            v = idx_v[pl.ds(o + k * 16, 16)]
            idx2_v[pl.ds(k * 16, 16)] = lax.shift_right_logical(v, 1)
            base_v[pl.ds(k * 16, 16)] = lax.shift_left(
                lax.bitwise_and(v, 1), 6)
        pltpu.async_copy(table_hbm.at[idx2_v], pairs_b, sem).wait()
        hb = tuple(base_v[pl.ds(k * 16, 16)] for k in range(8))

        @functools.partial(plsc.parallel_loop, 0, DIM, unroll=1)
        def d_body(d):
            vals = [plsc.load_gather(pairs_b, [jrow[jb], hb[jb] + d])
                    for jb in range(8)]
            for jb in range(8):
                out_b[d, pl.ds(jb * 16, 16)] = vals[jb]

        pltpu.sync_copy(
            out_b, out_hbm.at[l, :, pl.ds(pl.multiple_of(bt * 128, 128), 128)])
        return 0

    lax.fori_loop(0, UNITS_W, unit, 0)


@jax.jit
def _emb2(xt_flat, table_t, tail_p):
    mesh = plsc.VectorSubcoreMesh(core_axis_name="c", subcore_axis_name="s")
    tr = functools.partial(
        pl.kernel,
        mesh=mesh,
        out_type=jax.ShapeDtypeStruct((V // 2, 2 * DIM), jnp.float32),
        scratch_types=[
            pltpu.VMEM((DIM, CA), jnp.float32),
            pltpu.VMEM((CA // 2, 2 * DIM), jnp.float32),
            pltpu.VMEM((TAIL // 2, 2 * DIM), jnp.float32),
            pltpu.SemaphoreType.DMA,
        ],
        compiler_params=pltpu.CompilerParams(needs_layout_passes=False),
    )(_tr_body)
    table_rm = tr(table_t, tail_p)

    ga = functools.partial(
        pl.kernel,
        mesh=mesh,
        out_type=jax.ShapeDtypeStruct((L, DIM, B), jnp.float32),
        scratch_types=[
            pltpu.VMEM((N_W,), jnp.int32),
            pltpu.VMEM((128,), jnp.int32),
            pltpu.VMEM((128,), jnp.int32),
            pltpu.VMEM((128, 2 * DIM), jnp.float32),
            pltpu.VMEM((DIM, 128), jnp.float32),
            pltpu.SemaphoreType.DMA,
        ],
        compiler_params=pltpu.CompilerParams(needs_layout_passes=False),
    )(_ga_body)
    return ga(xt_flat, table_rm)


def kernel(x, table):
    xt_flat = x.T.reshape(N).astype(jnp.int32)
    tail_p = table[V_FULL:].reshape(TAIL // 2, 2 * DIM)
    out_p = _emb2(xt_flat, table.T, tail_p)
    return jnp.transpose(out_p, (2, 0, 1))


# speedup vs baseline: 4.3978x; 1.0785x over previous
import functools

import jax
import jax.numpy as jnp
from jax import lax
from jax.experimental import pallas as pl
from jax.experimental.pallas import tpu as pltpu
from jax.experimental.pallas import tpu_sc as plsc

B = 4096
L = 200
DIM = 64
V = 1000000

NC = 2
NS = 16
NW = NC * NS

V_FULL = 999936
CA = 512
N_CHUNKS = V_FULL // CA
TAIL = V - V_FULL


def _tr_body(tT_hbm, tail_hbm, out_hbm, in_b, ob, tail_b, sem):
    wid = lax.axis_index("s") * NC + lax.axis_index("c")
    n_chunks = jnp.where(wid < N_CHUNKS - (N_CHUNKS // NW) * NW,
                         N_CHUNKS // NW + 1, N_CHUNKS // NW)
    lane = lax.iota(jnp.int32, 16)

    rowpat = tuple(lax.rem(cb * 16 + lane, 64) for cb in range(8))
    colpat = tuple(lax.div(cb * 16 + lane, 64) for cb in range(8))

    def chunk(i, _):
        c0 = pl.multiple_of((wid + i * NW) * CA, 128)
        pltpu.sync_copy(tT_hbm.at[:, pl.ds(c0, CA)], in_b)

        @functools.partial(plsc.parallel_loop, 0, CA // 2, unroll=1)
        def p_body(p):
            twop = p * 2
            vals = [plsc.load_gather(in_b, [rowpat[cb], colpat[cb] + twop])
                    for cb in range(8)]
            for cb in range(8):
                ob[p, pl.ds(cb * 16, 16)] = vals[cb]

        pltpu.sync_copy(ob, out_hbm.at[pl.ds(lax.div(c0, 2), CA // 2)])
        return 0

    lax.fori_loop(0, n_chunks, chunk, 0)

    @pl.when(wid == 0)
    def _():
        pltpu.sync_copy(tail_hbm, tail_b)
        pltpu.sync_copy(tail_b, out_hbm.at[pl.ds(V_FULL // 2, TAIL // 2)])


N = B * L
N_W = N // NW
UNITS_W = N_W // 128


def _ga_body(idx_hbm, table_hbm, out_hbm, idx_v, idx2_v, base_v,
             pairs_b, out_b, sem):
    wid = lax.axis_index("s") * NC + lax.axis_index("c")
    base = wid * N_W
    pltpu.sync_copy(idx_hbm.at[pl.ds(base, N_W)], idx_v)
    lane = lax.iota(jnp.int32, 16)
    jrow = tuple(k * 16 + lane for k in range(8))

    def unit(u_local, _):
        u = wid * UNITS_W + u_local
        l = lax.div(u, 32)
        bt = lax.rem(u, 32)
        o = pl.multiple_of(u_local * 128, 128)
        for k in range(8):
            v = idx_v[pl.ds(o + k * 16, 16)]
            idx2_v[pl.ds(k * 16, 16)] = lax.shift_right_logical(v, 1)
            base_v[pl.ds(k * 16, 16)] = lax.shift_left(
                lax.bitwise_and(v, 1), 6)
        pltpu.async_copy(table_hbm.at[idx2_v], pairs_b, sem).wait()
        hb = tuple(base_v[pl.ds(k * 16, 16)] for k in range(8))

        @functools.partial(plsc.parallel_loop, 0, DIM, unroll=1)
        def d_body(d):
            vals = [plsc.load_gather(pairs_b, [jrow[jb], hb[jb] + d])
                    for jb in range(8)]
            for jb in range(8):
                out_b[d, pl.ds(jb * 16, 16)] = vals[jb]

        pltpu.sync_copy(
            out_b, out_hbm.at[l, :, pl.ds(pl.multiple_of(bt * 128, 128), 128)])
        return 0

    lax.fori_loop(0, UNITS_W, unit, 0)


@jax.jit
def _emb2(xt_flat, table_t, tail_p):
    mesh = plsc.VectorSubcoreMesh(core_axis_name="c", subcore_axis_name="s")
    tr = functools.partial(
        pl.kernel,
        mesh=mesh,
        out_type=jax.ShapeDtypeStruct((V // 2, 2 * DIM), jnp.float32),
        scratch_types=[
            pltpu.VMEM((DIM, CA), jnp.float32),
            pltpu.VMEM((CA // 2, 2 * DIM), jnp.float32),
            pltpu.VMEM((TAIL // 2, 2 * DIM), jnp.float32),
            pltpu.SemaphoreType.DMA,
        ],
        compiler_params=pltpu.CompilerParams(needs_layout_passes=False),
    )(_tr_body)
    table_rm = tr(table_t, tail_p)

    ga = functools.partial(
        pl.kernel,
        mesh=mesh,
        out_type=jax.ShapeDtypeStruct((L, DIM, B), jnp.float32),
        scratch_types=[
            pltpu.VMEM((N_W,), jnp.int32),
            pltpu.VMEM((128,), jnp.int32),
            pltpu.VMEM((128,), jnp.int32),
            pltpu.VMEM((128, 2 * DIM), jnp.float32),
            pltpu.VMEM((DIM, 128), jnp.float32),
            pltpu.SemaphoreType.DMA,
        ],
        compiler_params=pltpu.CompilerParams(needs_layout_passes=False),
    )(_ga_body)
    return ga(xt_flat, table_rm)


def kernel(x, table):
    xt_flat = x.T.reshape(N).astype(jnp.int32)
    tail_p = table[V_FULL:].reshape(TAIL // 2, 2 * DIM)
    out_p = _emb2(xt_flat, table.T, tail_p)
    return jnp.transpose(out_p, (2, 0, 1))
